# Initial kernel scaffold; baseline (speedup 1.0000x reference)
import jax
import jax.numpy as jnp
from jax.experimental import pallas as pl


def kernel(label_2d, vertex_pred, extents, poses, meta_data):
    raise NotImplementedError("write your pallas kernel here")


# speedup vs baseline: 3.9613x; 3.9613x over previous
import functools

import jax
import jax.numpy as jnp
from jax import lax
from jax.experimental import pallas as pl
from jax.experimental.pallas import tpu as pltpu
from jax.experimental.pallas import tpu_sc as plsc

C = 22
SKIP = 4
STEPS = 64
STEP_LEN = 8.0
H, W = 480, 640
HW = H * W
HS, WS = H // SKIP, W // SKIP
N = HS * WS
NR, NL = 150, 128
V = N * STEPS
ROWS = V // 128

NCLS = 6
NPASS = 2
ACC_MAIN = NCLS * HW
GARB_MASK = 32767
NWORDS = ACC_MAIN + 40960
ZCH = 2560
ZPT = NWORDS // 16
CH_ROWS = 20
CHUNK = CH_ROWS * 128
ROWS_PT = ROWS // 16
NCHUNK = ROWS_PT // CH_ROWS
STRIPE = HW // 16
BCH = 1920
NBCH = STRIPE // BCH
BIG = 2 ** 30


def _votes_body(vp_ref, lab_ref, bins_ref, wgt_ref, cnt_ref, dsum_ref):
    lab = lab_ref[...]
    dx = jnp.zeros((NR, NL), jnp.float32)
    dy = jnp.zeros((NR, NL), jnp.float32)
    for c in range(C):
        m = lab == c
        mf = m.astype(jnp.float32)
        cnt_ref[0, c] = jnp.sum(mf)
        dsum_ref[0, c] = jnp.sum(jnp.where(m, vp_ref[3 * c + 2], 0.0))
        dx = dx + jnp.where(m, vp_ref[3 * c + 0], 0.0)
        dy = dy + jnp.where(m, vp_ref[3 * c + 1], 0.0)
    nrm = jnp.sqrt(dx * dx + dy * dy) + 1e-6
    ux = dx / nrm
    uy = dy / nrm
    r = lax.broadcasted_iota(jnp.int32, (NR, NL), 0)
    l = lax.broadcasted_iota(jnp.int32, (NR, NL), 1)
    p = r * NL + l
    xs = p % WS
    ys = p // WS
    px = (xs * SKIP).astype(jnp.float32)
    py = (ys * SKIP).astype(jnp.float32)
    labpos = lab > 0
    lab_hw = lab * HW
    for s in range(STEPS):
        t = (s + 1) * STEP_LEN
        cx = px + t * ux
        cy = py + t * uy
        cxr = jnp.clip(jnp.round(cx), 0.0, W - 1.0)
        cyr = jnp.clip(jnp.round(cy), 0.0, H - 1.0)
        wv = (1.0 - jnp.abs(cx - cxr)) * (1.0 - jnp.abs(cy - cyr))
        valid = (cx >= 0) & (cx <= W - 1) & (cy >= 0) & (cy <= H - 1) & labpos
        wv = jnp.clip(wv, 0.0, 1.0) * valid.astype(jnp.float32)
        bins_ref[s] = lab_hw + cyr.astype(jnp.int32) * W + cxr.astype(jnp.int32)
        wgt_ref[s] = wv


_votes_call = pl.pallas_call(
    _votes_body,
    out_shape=[
        jax.ShapeDtypeStruct((STEPS, NR, NL), jnp.int32),
        jax.ShapeDtypeStruct((STEPS, NR, NL), jnp.float32),
        jax.ShapeDtypeStruct((1, C), jnp.float32),
        jax.ShapeDtypeStruct((1, C), jnp.float32),
    ],
    out_specs=[
        pl.BlockSpec(memory_space=pltpu.VMEM),
        pl.BlockSpec(memory_space=pltpu.VMEM),
        pl.BlockSpec(memory_space=pltpu.SMEM),
        pl.BlockSpec(memory_space=pltpu.SMEM),
    ],
)


def _sc_vote_body(bins_hbm, wgt_hbm, val_out, idx_out,
                  bins_v, wgt_v, idx_v, zbuf, mbuf, val_v, idx16_v, acc):
    core = lax.axis_index("c")
    sid = lax.axis_index("s")
    lane = lax.iota(jnp.int32, 16)

    def zb(i, _):
        zbuf[pl.ds(i * 16, 16)] = jnp.zeros((16,), jnp.float32)
        return 0

    lax.fori_loop(0, ZCH // 16, zb, 0)

    for p in range(NPASS):
        base = 1 + p * (2 * NCLS) + core * NCLS
        base_off = base * HW

        def zero_body(k, _):
            pltpu.sync_copy(zbuf, acc.at[pl.ds(sid * ZPT + k * ZCH, ZCH)])
            return 0

        lax.fori_loop(0, ZPT // ZCH, zero_body, 0)
        plsc.subcore_barrier()

        def sc_body(k, _):
            v0 = sid * (ROWS_PT * 128) + k * CHUNK
            pltpu.sync_copy(bins_hbm.at[pl.ds(v0, CHUNK)], bins_v)
            pltpu.sync_copy(wgt_hbm.at[pl.ds(v0, CHUNK)], wgt_v)

            def vb(i, _):
                b = bins_v[pl.ds(i * 16, 16)]
                rel = b - base_off
                ok = (rel >= 0) & (rel < ACC_MAIN)
                g = ACC_MAIN + (
                    (k * CHUNK + i * 16 + sid * 2048 + lane) & GARB_MASK)
                idx_v[pl.ds(i * 16, 16)] = jnp.where(ok, rel, g)
                return 0

            lax.fori_loop(0, CHUNK // 16, vb, 0)
            pltpu.sync_copy(wgt_v, acc.at[idx_v], add=True)
            return 0

        lax.fori_loop(0, NCHUNK, sc_body, 0)
        plsc.subcore_barrier()

        vinit = jnp.full((16,), -1.0, jnp.float32)
        iinit = jnp.zeros((16,), jnp.int32)
        for j in range(NCLS):
            def bch(kb, carry):
                mx0, mi0 = carry
                off = j * HW + sid * STRIPE + kb * BCH
                pltpu.sync_copy(acc.at[pl.ds(off, BCH)], mbuf)

                def vb2(i, c2):
                    mx, mi = c2
                    v = mbuf[pl.ds(i * 16, 16)]
                    gi = sid * STRIPE + kb * BCH + i * 16 + lane
                    upd = v > mx
                    return (jnp.where(upd, v, mx), jnp.where(upd, gi, mi))

                return lax.fori_loop(0, BCH // 16, vb2, (mx0, mi0))

            mx, mi = lax.fori_loop(0, NBCH, bch, (vinit, iinit))
            val_v[pl.ds(j * 16, 16)] = mx
            idx16_v[pl.ds(j * 16, 16)] = mi
        row = (p * 2 + core) * 16 + sid
        pltpu.sync_copy(val_v, val_out.at[row])
        pltpu.sync_copy(idx16_v, idx_out.at[row])
        plsc.subcore_barrier()


def _epi_body(val_ref, idx_ref, cnt_ref, dsum_ref, ext_ref, poses_ref,
              meta_ref, out_ref):
    vmax = [jnp.float32(0.0)] * C
    amax = [jnp.int32(0)] * C
    for q in range(4):
        p, co = q // 2, q % 2
        for sl in range(NCLS):
            cl = 1 + p * (2 * NCLS) + co * NCLS + sl
            if cl < C:
                blk = val_ref[pl.ds(q * 16, 16), pl.ds(sl * 16, 16)]
                ibk = idx_ref[pl.ds(q * 16, 16), pl.ds(sl * 16, 16)]
                mv = jnp.max(blk)
                ai = jnp.min(jnp.where(blk == mv, ibk, BIG))
                vmax[cl] = mv
                amax[cl] = ai
    fx = meta_ref[0, 0] * 500.0 + 500.0
    scores = []
    for cl in range(C):
        cnt = cnt_ref[0, cl]
        valid = ((vmax[cl] > 1.0) & (cnt > 500.0)
                 & (vmax[cl] / (cnt + 1.0) > 0.001))
        scores.append(vmax[cl] * valid.astype(jnp.float32))
    tot = scores[0]
    for cl in range(1, C):
        tot = tot + scores[cl]
    for cl in range(C):
        cnt = cnt_ref[0, cl]
        depth = dsum_ref[0, cl] / (cnt + 1e-6)
        e0 = ext_ref[cl, 0]
        e1 = ext_ref[cl, 1]
        e2 = ext_ref[cl, 2]
        diam = jnp.sqrt(e0 * e0 + e1 * e1 + e2 * e2 + 1e-8)
        scale = fx * diam / (jnp.abs(depth) + 0.1)
        cx0 = (amax[cl] % W).astype(jnp.float32)
        cy0 = (amax[cl] // W).astype(jnp.float32)
        out_ref[0, cl, 0] = jnp.float32(0.0)
        out_ref[0, cl, 1] = jnp.float32(float(cl))
        out_ref[0, cl, 2] = cx0 - scale * 0.5
        out_ref[0, cl, 3] = cy0 - scale * 0.5
        out_ref[0, cl, 4] = cx0 + scale * 0.5
        out_ref[0, cl, 5] = cy0 + scale * 0.5
        out_ref[0, cl, 6] = scores[cl]
        pw = scores[cl] / (tot + 1.0)
        for k in range(13):
            out_ref[0, cl, 7 + k] = poses_ref[cl, k] * pw


_epi_call = pl.pallas_call(
    _epi_body,
    out_shape=jax.ShapeDtypeStruct((1, C, 20), jnp.float32),
    in_specs=[
        pl.BlockSpec(memory_space=pltpu.VMEM),
        pl.BlockSpec(memory_space=pltpu.VMEM),
        pl.BlockSpec(memory_space=pltpu.SMEM),
        pl.BlockSpec(memory_space=pltpu.SMEM),
        pl.BlockSpec(memory_space=pltpu.SMEM),
        pl.BlockSpec(memory_space=pltpu.SMEM),
        pl.BlockSpec(memory_space=pltpu.SMEM),
    ],
    out_specs=pl.BlockSpec(memory_space=pltpu.SMEM),
)


@functools.cache
def _sc_vote_call():
    return pl.kernel(
        _sc_vote_body,
        out_type=[
            jax.ShapeDtypeStruct((4 * 16, NCLS * 16), jnp.float32),
            jax.ShapeDtypeStruct((4 * 16, NCLS * 16), jnp.int32),
        ],
        mesh=plsc.VectorSubcoreMesh(core_axis_name="c", subcore_axis_name="s"),
        scratch_types=[
            pltpu.VMEM((CHUNK,), jnp.int32),
            pltpu.VMEM((CHUNK,), jnp.float32),
            pltpu.VMEM((CHUNK,), jnp.int32),
            pltpu.VMEM((ZCH,), jnp.float32),
            pltpu.VMEM((BCH,), jnp.float32),
            pltpu.VMEM((NCLS * 16,), jnp.float32),
            pltpu.VMEM((NCLS * 16,), jnp.int32),
            pltpu.VMEM_SHARED((NWORDS,), jnp.float32),
        ],
    )


def kernel(label_2d, vertex_pred, extents, poses, meta_data):
    lab_s = label_2d[0, ::SKIP, ::SKIP].astype(jnp.int32).reshape(NR, NL)
    vp_s = (vertex_pred[0].reshape(C, 3, H, W)[:, :, ::SKIP, ::SKIP]
            .astype(jnp.float32).reshape(3 * C, NR, NL))
    bins, wgt, cnt, dsum = _votes_call(vp_s, lab_s)
    val, idx = _sc_vote_call()(bins.reshape(V), wgt.reshape(V))
    return _epi_call(val, idx, cnt, dsum, extents.astype(jnp.float32),
                     poses.astype(jnp.float32), meta_data.astype(jnp.float32))


# speedup vs baseline: 5.0296x; 1.2697x over previous
import functools

import jax
import jax.numpy as jnp
from jax import lax
from jax.experimental import pallas as pl
from jax.experimental.pallas import tpu as pltpu
from jax.experimental.pallas import tpu_sc as plsc

C = 22
SKIP = 4
STEPS = 64
STEP_LEN = 8.0
H, W = 480, 640
HW = H * W
HS, WS = H // SKIP, W // SKIP
N = HS * WS
NR, NL = 152, 128
NP = NR * NL
V = NP * STEPS

NCLS = 6
NPASS = 2
ACC_MAIN = NCLS * HW
GARB_MASK = 32767
NWORDS = ACC_MAIN + 40960
ZCH = 2560
ZPT = NWORDS // 16
CHUNK = 2432
VPT = V // 16
NCHUNK = VPT // CHUNK
STRIPE = HW // 16
BCH = 1920
NBCH = STRIPE // BCH
BIG = 2 ** 30


def _votes_body(vp_ref, lab_ref, bins_ref, wgt_ref, cnt_ref, dsum_ref):
    lab = lab_ref[...]
    dx = jnp.zeros((NR, NL), jnp.float32)
    dy = jnp.zeros((NR, NL), jnp.float32)
    for c in range(C):
        m = lab == c
        mf = m.astype(jnp.float32)
        cnt_ref[0, c] = jnp.sum(mf)
        dsum_ref[0, c] = jnp.sum(jnp.where(m, vp_ref[3 * c + 2], 0.0))
        dx = dx + jnp.where(m, vp_ref[3 * c + 0], 0.0)
        dy = dy + jnp.where(m, vp_ref[3 * c + 1], 0.0)
    nrm = jnp.sqrt(dx * dx + dy * dy) + 1e-6
    ux = dx / nrm
    uy = dy / nrm
    r = lax.broadcasted_iota(jnp.int32, (NR, NL), 0)
    l = lax.broadcasted_iota(jnp.int32, (NR, NL), 1)
    p = r * NL + l
    xs = p % WS
    ys = p // WS
    px = (xs * SKIP).astype(jnp.float32)
    py = (ys * SKIP).astype(jnp.float32)
    labpos = lab > 0
    lab_hw = lab * HW
    for s in range(STEPS):
        t = (s + 1) * STEP_LEN
        cx = px + t * ux
        cy = py + t * uy
        cxr = jnp.clip(jnp.round(cx), 0.0, W - 1.0)
        cyr = jnp.clip(jnp.round(cy), 0.0, H - 1.0)
        wv = (1.0 - jnp.abs(cx - cxr)) * (1.0 - jnp.abs(cy - cyr))
        valid = (cx >= 0) & (cx <= W - 1) & (cy >= 0) & (cy <= H - 1) & labpos
        wv = jnp.clip(wv, 0.0, 1.0) * valid.astype(jnp.float32)
        bins_ref[s] = lab_hw + cyr.astype(jnp.int32) * W + cxr.astype(jnp.int32)
        wgt_ref[s] = wv


_votes_call = pl.pallas_call(
    _votes_body,
    out_shape=[
        jax.ShapeDtypeStruct((STEPS, NR, NL), jnp.int32),
        jax.ShapeDtypeStruct((STEPS, NR, NL), jnp.float32),
        jax.ShapeDtypeStruct((1, C), jnp.float32),
        jax.ShapeDtypeStruct((1, C), jnp.float32),
    ],
    out_specs=[
        pl.BlockSpec(memory_space=pltpu.VMEM),
        pl.BlockSpec(memory_space=pltpu.VMEM),
        pl.BlockSpec(memory_space=pltpu.SMEM),
        pl.BlockSpec(memory_space=pltpu.SMEM),
    ],
)


def _sc_vote_body(bins_hbm, wgt_hbm, val_out, idx_out,
                  bins_v, wgt_v, idx_v, zbuf, mbuf, val_v, idx16_v, acc):
    core = lax.axis_index("c")
    sid = lax.axis_index("s")
    lane = lax.iota(jnp.int32, 16)

    def zb(i, _):
        zbuf[pl.ds(i * 16, 16)] = jnp.zeros((16,), jnp.float32)
        return 0

    lax.fori_loop(0, ZCH // 16, zb, 0)

    for p in range(NPASS):
        base = 1 + p * (2 * NCLS) + core * NCLS
        base_off = base * HW

        def zero_body(k, _):
            pltpu.sync_copy(zbuf, acc.at[pl.ds(sid * ZPT + k * ZCH, ZCH)])
            return 0

        lax.fori_loop(0, ZPT // ZCH, zero_body, 0)
        plsc.subcore_barrier()

        def sc_body(k, _):
            v0 = sid * VPT + k * CHUNK
            pltpu.sync_copy(bins_hbm.at[pl.ds(v0, CHUNK)], bins_v)
            pltpu.sync_copy(wgt_hbm.at[pl.ds(v0, CHUNK)], wgt_v)

            def vb(i, _):
                b = bins_v[pl.ds(i * 16, 16)]
                rel = b - base_off
                ok = (rel >= 0) & (rel < ACC_MAIN)
                g = ACC_MAIN + (
                    (k * CHUNK + i * 16 + sid * 2048 + lane) & GARB_MASK)
                idx_v[pl.ds(i * 16, 16)] = jnp.where(ok, rel, g)
                return 0

            lax.fori_loop(0, CHUNK // 16, vb, 0)
            pltpu.sync_copy(wgt_v, acc.at[idx_v], add=True)
            return 0

        lax.fori_loop(0, NCHUNK, sc_body, 0)
        plsc.subcore_barrier()

        vinit = jnp.full((16,), -1.0, jnp.float32)
        iinit = jnp.zeros((16,), jnp.int32)
        for j in range(NCLS):
            def bch(kb, carry):
                mx0, mi0 = carry
                off = j * HW + sid * STRIPE + kb * BCH
                pltpu.sync_copy(acc.at[pl.ds(off, BCH)], mbuf)

                def vb2(i, c2):
                    mx, mi = c2
                    v = mbuf[pl.ds(i * 16, 16)]
                    gi = sid * STRIPE + kb * BCH + i * 16 + lane
                    upd = v > mx
                    return (jnp.where(upd, v, mx), jnp.where(upd, gi, mi))

                return lax.fori_loop(0, BCH // 16, vb2, (mx0, mi0))

            mx, mi = lax.fori_loop(0, NBCH, bch, (vinit, iinit))
            val_v[pl.ds(j * 16, 16)] = mx
            idx16_v[pl.ds(j * 16, 16)] = mi
        row = (p * 2 + core) * 16 + sid
        pltpu.sync_copy(val_v, val_out.at[row])
        pltpu.sync_copy(idx16_v, idx_out.at[row])
        plsc.subcore_barrier()


def _epi_body(val_ref, idx_ref, cnt_ref, dsum_ref, ext_ref, poses_ref,
              meta_ref, out_ref):
    vmax = [jnp.float32(0.0)] * C
    amax = [jnp.int32(0)] * C
    for q in range(4):
        p, co = q // 2, q % 2
        for sl in range(NCLS):
            cl = 1 + p * (2 * NCLS) + co * NCLS + sl
            if cl < C:
                blk = val_ref[pl.ds(q * 16, 16), pl.ds(sl * 16, 16)]
                ibk = idx_ref[pl.ds(q * 16, 16), pl.ds(sl * 16, 16)]
                mv = jnp.max(blk)
                ai = jnp.min(jnp.where(blk == mv, ibk, BIG))
                vmax[cl] = mv
                amax[cl] = ai
    fx = meta_ref[0, 0] * 500.0 + 500.0
    scores = []
    for cl in range(C):
        cnt = cnt_ref[0, cl]
        valid = ((vmax[cl] > 1.0) & (cnt > 500.0)
                 & (vmax[cl] / (cnt + 1.0) > 0.001))
        scores.append(vmax[cl] * valid.astype(jnp.float32))
    tot = scores[0]
    for cl in range(1, C):
        tot = tot + scores[cl]
    for cl in range(C):
        cnt = cnt_ref[0, cl]
        depth = dsum_ref[0, cl] / (cnt + 1e-6)
        e0 = ext_ref[cl, 0]
        e1 = ext_ref[cl, 1]
        e2 = ext_ref[cl, 2]
        diam = jnp.sqrt(e0 * e0 + e1 * e1 + e2 * e2 + 1e-8)
        scale = fx * diam / (jnp.abs(depth) + 0.1)
        cx0 = (amax[cl] % W).astype(jnp.float32)
        cy0 = (amax[cl] // W).astype(jnp.float32)
        out_ref[0, cl, 0] = jnp.float32(0.0)
        out_ref[0, cl, 1] = jnp.float32(float(cl))
        out_ref[0, cl, 2] = cx0 - scale * 0.5
        out_ref[0, cl, 3] = cy0 - scale * 0.5
        out_ref[0, cl, 4] = cx0 + scale * 0.5
        out_ref[0, cl, 5] = cy0 + scale * 0.5
        out_ref[0, cl, 6] = scores[cl]
        pw = scores[cl] / (tot + 1.0)
        for k in range(13):
            out_ref[0, cl, 7 + k] = poses_ref[cl, k] * pw


_epi_call = pl.pallas_call(
    _epi_body,
    out_shape=jax.ShapeDtypeStruct((1, C, 20), jnp.float32),
    in_specs=[
        pl.BlockSpec(memory_space=pltpu.VMEM),
        pl.BlockSpec(memory_space=pltpu.VMEM),
        pl.BlockSpec(memory_space=pltpu.SMEM),
        pl.BlockSpec(memory_space=pltpu.SMEM),
        pl.BlockSpec(memory_space=pltpu.SMEM),
        pl.BlockSpec(memory_space=pltpu.SMEM),
        pl.BlockSpec(memory_space=pltpu.SMEM),
    ],
    out_specs=pl.BlockSpec(memory_space=pltpu.SMEM),
)


@functools.cache
def _sc_vote_call():
    return pl.kernel(
        _sc_vote_body,
        out_type=[
            jax.ShapeDtypeStruct((4 * 16, NCLS * 16), jnp.float32),
            jax.ShapeDtypeStruct((4 * 16, NCLS * 16), jnp.int32),
        ],
        mesh=plsc.VectorSubcoreMesh(core_axis_name="c", subcore_axis_name="s"),
        scratch_types=[
            pltpu.VMEM((CHUNK,), jnp.int32),
            pltpu.VMEM((CHUNK,), jnp.float32),
            pltpu.VMEM((CHUNK,), jnp.int32),
            pltpu.VMEM((ZCH,), jnp.float32),
            pltpu.VMEM((BCH,), jnp.float32),
            pltpu.VMEM((NCLS * 16,), jnp.float32),
            pltpu.VMEM((NCLS * 16,), jnp.int32),
            pltpu.VMEM_SHARED((NWORDS,), jnp.float32),
        ],
    )


def kernel(label_2d, vertex_pred, extents, poses, meta_data):
    lab_s = jnp.pad(
        label_2d[0, ::SKIP, ::SKIP].astype(jnp.int32).reshape(N),
        (0, NP - N), constant_values=C).reshape(NR, NL)
    vp_r = vertex_pred[0].reshape(3 * C, H, W)[:, ::SKIP, :]
    vp_s = jnp.pad(
        vp_r[:, :, ::SKIP].astype(jnp.float32).reshape(3 * C, N),
        ((0, 0), (0, NP - N))).reshape(3 * C, NR, NL)
    bins, wgt, cnt, dsum = _votes_call(vp_s, lab_s)
    val, idx = _sc_vote_call()(bins.reshape(V), wgt.reshape(V))
    return _epi_call(val, idx, cnt, dsum, extents.astype(jnp.float32),
                     poses.astype(jnp.float32), meta_data.astype(jnp.float32))


# speedup vs baseline: 8.2726x; 1.6448x over previous
import functools

import jax
import jax.numpy as jnp
from jax import lax
from jax.experimental import pallas as pl
from jax.experimental.pallas import tpu as pltpu
from jax.experimental.pallas import tpu_sc as plsc

C = 22
SKIP = 4
STEPS = 64
STEP_LEN = 8.0
H, W = 480, 640
HW = H * W
HS, WS = H // SKIP, W // SKIP
N = HS * WS
NR, NL = 152, 128
NP = NR * NL
V = NP * STEPS

NCLS = 6
NPASS = 2
ACC_MAIN = NCLS * HW
GARB_MASK = 32767
NWORDS = ACC_MAIN + 40960
ZCH = 2560
ZPT = NWORDS // 16
CHUNK = 2432
VPT = V // 16
NCHUNK = VPT // CHUNK
STRIPE = HW // 16
BCH = 1920
NBCH = STRIPE // BCH
BIG = 2 ** 30


def _gather_body(vp_ref, lab_ref, uvd_ref):
    lab = lab_ref[...]
    dx = jnp.zeros((8, W), jnp.float32)
    dy = jnp.zeros((8, W), jnp.float32)
    dz = jnp.zeros((8, W), jnp.float32)
    for c in range(C):
        m = lab == c
        dx = dx + jnp.where(m, vp_ref[3 * c + 0], 0.0)
        dy = dy + jnp.where(m, vp_ref[3 * c + 1], 0.0)
        dz = dz + jnp.where(m, vp_ref[3 * c + 2], 0.0)
    nrm = jnp.sqrt(dx * dx + dy * dy) + 1e-6
    uvd_ref[0] = dx / nrm
    uvd_ref[1] = dy / nrm
    uvd_ref[2] = dz


_gather_call = pl.pallas_call(
    _gather_body,
    grid=(H // 8,),
    in_specs=[
        pl.BlockSpec((3 * C, 8, W), lambda g: (0, g, 0)),
        pl.BlockSpec((8, W), lambda g: (g, 0)),
    ],
    out_specs=pl.BlockSpec((3, 8, W), lambda g: (0, g, 0)),
    out_shape=jax.ShapeDtypeStruct((3, H, W), jnp.float32),
)


def _votes_body(uvd_ref, lab_ref, bins_ref, wgt_ref, cnt_ref, dsum_ref):
    lab = lab_ref[...]
    ux = uvd_ref[0]
    uy = uvd_ref[1]
    dz = uvd_ref[2]
    for c in range(C):
        m = lab == c
        cnt_ref[0, c] = jnp.sum(m.astype(jnp.float32))
        dsum_ref[0, c] = jnp.sum(jnp.where(m, dz, 0.0))
    r = lax.broadcasted_iota(jnp.int32, (NR, NL), 0)
    l = lax.broadcasted_iota(jnp.int32, (NR, NL), 1)
    p = r * NL + l
    xs = p % WS
    ys = p // WS
    px = (xs * SKIP).astype(jnp.float32)
    py = (ys * SKIP).astype(jnp.float32)
    labpos = lab > 0
    lab_hw = lab * HW
    for s in range(STEPS):
        t = (s + 1) * STEP_LEN
        cx = px + t * ux
        cy = py + t * uy
        cxr = jnp.clip(jnp.round(cx), 0.0, W - 1.0)
        cyr = jnp.clip(jnp.round(cy), 0.0, H - 1.0)
        wv = (1.0 - jnp.abs(cx - cxr)) * (1.0 - jnp.abs(cy - cyr))
        valid = (cx >= 0) & (cx <= W - 1) & (cy >= 0) & (cy <= H - 1) & labpos
        wv = jnp.clip(wv, 0.0, 1.0) * valid.astype(jnp.float32)
        bins_ref[s] = lab_hw + cyr.astype(jnp.int32) * W + cxr.astype(jnp.int32)
        wgt_ref[s] = wv


_votes_call = pl.pallas_call(
    _votes_body,
    out_shape=[
        jax.ShapeDtypeStruct((STEPS, NR, NL), jnp.int32),
        jax.ShapeDtypeStruct((STEPS, NR, NL), jnp.float32),
        jax.ShapeDtypeStruct((1, C), jnp.float32),
        jax.ShapeDtypeStruct((1, C), jnp.float32),
    ],
    out_specs=[
        pl.BlockSpec(memory_space=pltpu.VMEM),
        pl.BlockSpec(memory_space=pltpu.VMEM),
        pl.BlockSpec(memory_space=pltpu.SMEM),
        pl.BlockSpec(memory_space=pltpu.SMEM),
    ],
)


def _sc_vote_body(bins_hbm, wgt_hbm, val_out, idx_out,
                  bins_v, wgt_v, idx_v, zbuf, mbuf, val_v, idx16_v, acc):
    core = lax.axis_index("c")
    sid = lax.axis_index("s")
    lane = lax.iota(jnp.int32, 16)

    def zb(i, _):
        zbuf[pl.ds(i * 16, 16)] = jnp.zeros((16,), jnp.float32)
        return 0

    lax.fori_loop(0, ZCH // 16, zb, 0)

    for p in range(NPASS):
        base = 1 + p * (2 * NCLS) + core * NCLS
        base_off = base * HW

        def zero_body(k, _):
            pltpu.sync_copy(zbuf, acc.at[pl.ds(sid * ZPT + k * ZCH, ZCH)])
            return 0

        lax.fori_loop(0, ZPT // ZCH, zero_body, 0)
        plsc.subcore_barrier()

        def sc_body(k, _):
            v0 = sid * VPT + k * CHUNK
            pltpu.sync_copy(bins_hbm.at[pl.ds(v0, CHUNK)], bins_v)
            pltpu.sync_copy(wgt_hbm.at[pl.ds(v0, CHUNK)], wgt_v)

            def vb(i, _):
                b = bins_v[pl.ds(i * 16, 16)]
                rel = b - base_off
                ok = (rel >= 0) & (rel < ACC_MAIN)
                g = ACC_MAIN + (
                    (k * CHUNK + i * 16 + sid * 2048 + lane) & GARB_MASK)
                idx_v[pl.ds(i * 16, 16)] = jnp.where(ok, rel, g)
                return 0

            lax.fori_loop(0, CHUNK // 16, vb, 0)
            pltpu.sync_copy(wgt_v, acc.at[idx_v], add=True)
            return 0

        lax.fori_loop(0, NCHUNK, sc_body, 0)
        plsc.subcore_barrier()

        vinit = jnp.full((16,), -1.0, jnp.float32)
        iinit = jnp.zeros((16,), jnp.int32)
        for j in range(NCLS):
            def bch(kb, carry):
                mx0, mi0 = carry
                off = j * HW + sid * STRIPE + kb * BCH
                pltpu.sync_copy(acc.at[pl.ds(off, BCH)], mbuf)

                def vb2(i, c2):
                    mx, mi = c2
                    v = mbuf[pl.ds(i * 16, 16)]
                    gi = sid * STRIPE + kb * BCH + i * 16 + lane
                    upd = v > mx
                    return (jnp.where(upd, v, mx), jnp.where(upd, gi, mi))

                return lax.fori_loop(0, BCH // 16, vb2, (mx0, mi0))

            mx, mi = lax.fori_loop(0, NBCH, bch, (vinit, iinit))
            val_v[pl.ds(j * 16, 16)] = mx
            idx16_v[pl.ds(j * 16, 16)] = mi
        row = (p * 2 + core) * 16 + sid
        pltpu.sync_copy(val_v, val_out.at[row])
        pltpu.sync_copy(idx16_v, idx_out.at[row])
        plsc.subcore_barrier()


def _epi_body(val_ref, idx_ref, cnt_ref, dsum_ref, ext_ref, poses_ref,
              meta_ref, out_ref):
    vmax = [jnp.float32(0.0)] * C
    amax = [jnp.int32(0)] * C
    for q in range(4):
        p, co = q // 2, q % 2
        for sl in range(NCLS):
            cl = 1 + p * (2 * NCLS) + co * NCLS + sl
            if cl < C:
                blk = val_ref[pl.ds(q * 16, 16), pl.ds(sl * 16, 16)]
                ibk = idx_ref[pl.ds(q * 16, 16), pl.ds(sl * 16, 16)]
                mv = jnp.max(blk)
                ai = jnp.min(jnp.where(blk == mv, ibk, BIG))
                vmax[cl] = mv
                amax[cl] = ai
    fx = meta_ref[0, 0] * 500.0 + 500.0
    scores = []
    for cl in range(C):
        cnt = cnt_ref[0, cl]
        valid = ((vmax[cl] > 1.0) & (cnt > 500.0)
                 & (vmax[cl] / (cnt + 1.0) > 0.001))
        scores.append(vmax[cl] * valid.astype(jnp.float32))
    tot = scores[0]
    for cl in range(1, C):
        tot = tot + scores[cl]
    for cl in range(C):
        cnt = cnt_ref[0, cl]
        depth = dsum_ref[0, cl] / (cnt + 1e-6)
        e0 = ext_ref[cl, 0]
        e1 = ext_ref[cl, 1]
        e2 = ext_ref[cl, 2]
        diam = jnp.sqrt(e0 * e0 + e1 * e1 + e2 * e2 + 1e-8)
        scale = fx * diam / (jnp.abs(depth) + 0.1)
        cx0 = (amax[cl] % W).astype(jnp.float32)
        cy0 = (amax[cl] // W).astype(jnp.float32)
        out_ref[0, cl, 0] = jnp.float32(0.0)
        out_ref[0, cl, 1] = jnp.float32(float(cl))
        out_ref[0, cl, 2] = cx0 - scale * 0.5
        out_ref[0, cl, 3] = cy0 - scale * 0.5
        out_ref[0, cl, 4] = cx0 + scale * 0.5
        out_ref[0, cl, 5] = cy0 + scale * 0.5
        out_ref[0, cl, 6] = scores[cl]
        pw = scores[cl] / (tot + 1.0)
        for k in range(13):
            out_ref[0, cl, 7 + k] = poses_ref[cl, k] * pw


_epi_call = pl.pallas_call(
    _epi_body,
    out_shape=jax.ShapeDtypeStruct((1, C, 20), jnp.float32),
    in_specs=[
        pl.BlockSpec(memory_space=pltpu.VMEM),
        pl.BlockSpec(memory_space=pltpu.VMEM),
        pl.BlockSpec(memory_space=pltpu.SMEM),
        pl.BlockSpec(memory_space=pltpu.SMEM),
        pl.BlockSpec(memory_space=pltpu.SMEM),
        pl.BlockSpec(memory_space=pltpu.SMEM),
        pl.BlockSpec(memory_space=pltpu.SMEM),
    ],
    out_specs=pl.BlockSpec(memory_space=pltpu.SMEM),
)


@functools.cache
def _sc_vote_call():
    return pl.kernel(
        _sc_vote_body,
        out_type=[
            jax.ShapeDtypeStruct((4 * 16, NCLS * 16), jnp.float32),
            jax.ShapeDtypeStruct((4 * 16, NCLS * 16), jnp.int32),
        ],
        mesh=plsc.VectorSubcoreMesh(core_axis_name="c", subcore_axis_name="s"),
        scratch_types=[
            pltpu.VMEM((CHUNK,), jnp.int32),
            pltpu.VMEM((CHUNK,), jnp.float32),
            pltpu.VMEM((CHUNK,), jnp.int32),
            pltpu.VMEM((ZCH,), jnp.float32),
            pltpu.VMEM((BCH,), jnp.float32),
            pltpu.VMEM((NCLS * 16,), jnp.float32),
            pltpu.VMEM((NCLS * 16,), jnp.int32),
            pltpu.VMEM_SHARED((NWORDS,), jnp.float32),
        ],
    )


def kernel(label_2d, vertex_pred, extents, poses, meta_data):
    lab_full = label_2d[0].astype(jnp.int32)
    lab_s = jnp.pad(
        lab_full[::SKIP, ::SKIP].reshape(N),
        (0, NP - N), constant_values=C).reshape(NR, NL)
    uvd = _gather_call(vertex_pred[0].reshape(3 * C, H, W).astype(jnp.float32),
                       lab_full)
    uvd_s = jnp.pad(
        uvd[:, ::SKIP, ::SKIP].reshape(3, N),
        ((0, 0), (0, NP - N))).reshape(3, NR, NL)
    bins, wgt, cnt, dsum = _votes_call(uvd_s, lab_s)
    val, idx = _sc_vote_call()(bins.reshape(V), wgt.reshape(V))
    return _epi_call(val, idx, cnt, dsum, extents.astype(jnp.float32),
                     poses.astype(jnp.float32), meta_data.astype(jnp.float32))


# speedup vs baseline: 11.4601x; 1.3853x over previous
import functools

import jax
import jax.numpy as jnp
from jax import lax
from jax.experimental import pallas as pl
from jax.experimental.pallas import tpu as pltpu
from jax.experimental.pallas import tpu_sc as plsc

C = 22
SKIP = 4
STEPS = 64
STEP_LEN = 8.0
H, W = 480, 640
HW = H * W
HS, WS = H // SKIP, W // SKIP
N = HS * WS
NR, NL = 152, 128
NP = NR * NL
V = NP * STEPS

NCLS = 6
NPASS = 2
ACC_MAIN = NCLS * HW
GARB_MASK = 8191
NWORDS = ACC_MAIN + 8192
ZCH = 1024
ZPT = NWORDS // 16
CHUNK = 2048
VPT = V // 16
NCHUNK = VPT // CHUNK
STRIPE = HW // 16
BCH = 800
NBCH = STRIPE // BCH
BIG = 2 ** 30


def _gather_body(vp_ref, lab_ref, uvd_ref):
    lab = lab_ref[...]
    dx = jnp.zeros((8, W), jnp.float32)
    dy = jnp.zeros((8, W), jnp.float32)
    dz = jnp.zeros((8, W), jnp.float32)
    for c in range(C):
        m = lab == c
        dx = dx + jnp.where(m, vp_ref[3 * c + 0], 0.0)
        dy = dy + jnp.where(m, vp_ref[3 * c + 1], 0.0)
        dz = dz + jnp.where(m, vp_ref[3 * c + 2], 0.0)
    nrm = jnp.sqrt(dx * dx + dy * dy) + 1e-6
    uvd_ref[0] = dx / nrm
    uvd_ref[1] = dy / nrm
    uvd_ref[2] = dz


_gather_call = pl.pallas_call(
    _gather_body,
    grid=(H // 8,),
    in_specs=[
        pl.BlockSpec((3 * C, 8, W), lambda g: (0, g, 0)),
        pl.BlockSpec((8, W), lambda g: (g, 0)),
    ],
    out_specs=pl.BlockSpec((3, 8, W), lambda g: (0, g, 0)),
    out_shape=jax.ShapeDtypeStruct((3, H, W), jnp.float32),
)


def _votes_body(uvd_ref, lab_ref, bins_ref, wgt_ref, cnt_ref, dsum_ref):
    lab = lab_ref[...]
    ux = uvd_ref[0]
    uy = uvd_ref[1]
    dz = uvd_ref[2]
    for c in range(C):
        m = lab == c
        cnt_ref[0, c] = jnp.sum(m.astype(jnp.float32))
        dsum_ref[0, c] = jnp.sum(jnp.where(m, dz, 0.0))
    r = lax.broadcasted_iota(jnp.int32, (NR, NL), 0)
    l = lax.broadcasted_iota(jnp.int32, (NR, NL), 1)
    p = r * NL + l
    xs = p % WS
    ys = p // WS
    px = (xs * SKIP).astype(jnp.float32)
    py = (ys * SKIP).astype(jnp.float32)
    labpos = lab > 0
    lab_hw = lab * HW
    for s in range(STEPS):
        t = (s + 1) * STEP_LEN
        cx = px + t * ux
        cy = py + t * uy
        cxr = jnp.clip(jnp.round(cx), 0.0, W - 1.0)
        cyr = jnp.clip(jnp.round(cy), 0.0, H - 1.0)
        wv = (1.0 - jnp.abs(cx - cxr)) * (1.0 - jnp.abs(cy - cyr))
        valid = (cx >= 0) & (cx <= W - 1) & (cy >= 0) & (cy <= H - 1) & labpos
        wv = jnp.clip(wv, 0.0, 1.0) * valid.astype(jnp.float32)
        bins_ref[s] = lab_hw + cyr.astype(jnp.int32) * W + cxr.astype(jnp.int32)
        wgt_ref[s] = wv


_votes_call = pl.pallas_call(
    _votes_body,
    out_shape=[
        jax.ShapeDtypeStruct((STEPS, NR, NL), jnp.int32),
        jax.ShapeDtypeStruct((STEPS, NR, NL), jnp.float32),
        jax.ShapeDtypeStruct((1, C), jnp.float32),
        jax.ShapeDtypeStruct((1, C), jnp.float32),
    ],
    out_specs=[
        pl.BlockSpec(memory_space=pltpu.VMEM),
        pl.BlockSpec(memory_space=pltpu.VMEM),
        pl.BlockSpec(memory_space=pltpu.SMEM),
        pl.BlockSpec(memory_space=pltpu.SMEM),
    ],
)


def _sc_vote_body(bins_hbm, wgt_hbm, val_out, idx_out,
                  bins_v0, bins_v1, wgt_v0, wgt_v1, idx_v,
                  zbuf, mbuf0, mbuf1, val_v, idx16_v, acc,
                  sem_in0, sem_in1, sem_sc, sem_z,
                  sem_mb0, sem_mb1):
    core = lax.axis_index("c")
    sid = lax.axis_index("s")
    lane = lax.iota(jnp.int32, 16)
    bufs = ((bins_v0, wgt_v0, sem_in0),
            (bins_v1, wgt_v1, sem_in1))
    mbufs = ((mbuf0, sem_mb0), (mbuf1, sem_mb1))

    def zb(i, _):
        zbuf[pl.ds(i * 16, 16)] = jnp.zeros((16,), jnp.float32)
        return 0

    lax.fori_loop(0, ZCH // 16, zb, 0)

    for p in range(NPASS):
        base = 1 + p * (2 * NCLS) + core * NCLS
        base_off = base * HW

        pltpu.async_copy(bins_hbm.at[pl.ds(sid * VPT, CHUNK)], bins_v0,
                         sem_in0)
        pltpu.async_copy(wgt_hbm.at[pl.ds(sid * VPT, CHUNK)], wgt_v0,
                         sem_in0)

        def zero_body(k, _):
            pltpu.async_copy(zbuf, acc.at[pl.ds(sid * ZPT + k * ZCH, ZCH)],
                             sem_z)
            return 0

        lax.fori_loop(0, ZPT // ZCH, zero_body, 0)

        def zero_drain(k, _):
            pltpu.make_async_copy(zbuf, acc.at[pl.ds(sid * ZPT, ZCH)],
                                  sem_z).wait()
            return 0

        lax.fori_loop(0, ZPT // ZCH, zero_drain, 0)
        plsc.subcore_barrier()

        def sc_body(k, _):
            for b in range(2):
                bv, wv, s_in = bufs[b]
                ov, ow, o_in = bufs[1 - b]
                kk = 2 * k + b
                v0 = sid * VPT + kk * CHUNK
                pltpu.make_async_copy(bins_hbm.at[pl.ds(v0, CHUNK)], bv,
                                      s_in).wait()
                pltpu.make_async_copy(wgt_hbm.at[pl.ds(v0, CHUNK)], wv,
                                      s_in).wait()

                @pl.when(kk >= 1)
                def _():
                    pltpu.make_async_copy(ow, acc.at[idx_v], sem_sc).wait()

                @pl.when(kk + 1 < NCHUNK)
                def _():
                    v1 = sid * VPT + (kk + 1) * CHUNK
                    pltpu.async_copy(bins_hbm.at[pl.ds(v1, CHUNK)], ov, o_in)
                    pltpu.async_copy(wgt_hbm.at[pl.ds(v1, CHUNK)], ow, o_in)

                def vb(i, _):
                    bq = bv[pl.ds(i * 16, 16)]
                    rel = bq - base_off
                    ok = (rel >= 0) & (rel < ACC_MAIN)
                    g = ACC_MAIN + (
                        (kk * CHUNK + i * 16 + sid * 2048 + lane) & GARB_MASK)
                    idx_v[pl.ds(i * 16, 16)] = jnp.where(ok, rel, g)
                    return 0

                lax.fori_loop(0, CHUNK // 16, vb, 0)
                pltpu.async_copy(wv, acc.at[idx_v], sem_sc, add=True)
            return 0

        lax.fori_loop(0, NCHUNK // 2, sc_body, 0)
        pltpu.make_async_copy(wgt_v1, acc.at[idx_v], sem_sc).wait()
        plsc.subcore_barrier()

        vinit = jnp.full((16,), -1.0, jnp.float32)
        iinit = jnp.zeros((16,), jnp.int32)
        for j in range(NCLS):
            base_b = j * HW + sid * STRIPE
            pltpu.async_copy(acc.at[pl.ds(base_b, BCH)], mbuf0, sem_mb0)

            def bch2(kb2, carry, base_b=base_b):
                mxc, mic = carry
                for b in range(2):
                    mb, s_mb = mbufs[b]
                    om, o_mb = mbufs[1 - b]
                    kb = 2 * kb2 + b
                    pltpu.make_async_copy(acc.at[pl.ds(base_b, BCH)], mb,
                                          s_mb).wait()

                    @pl.when(kb + 1 < NBCH)
                    def _(kb=kb, om=om, o_mb=o_mb):
                        pltpu.async_copy(
                            acc.at[pl.ds(base_b + (kb + 1) * BCH, BCH)],
                            om, o_mb)

                    def vb2(i, c2, kb=kb, mb=mb):
                        mx2, mi2 = c2
                        v = mb[pl.ds(i * 16, 16)]
                        gi = sid * STRIPE + kb * BCH + i * 16 + lane
                        upd = v > mx2
                        return (jnp.where(upd, v, mx2),
                                jnp.where(upd, gi, mi2))

                    mxc, mic = lax.fori_loop(0, BCH // 16, vb2, (mxc, mic))
                return (mxc, mic)

            mx, mi = lax.fori_loop(0, NBCH // 2, bch2, (vinit, iinit))
            val_v[pl.ds(j * 16, 16)] = mx
            idx16_v[pl.ds(j * 16, 16)] = mi
        row = (p * 2 + core) * 16 + sid
        pltpu.sync_copy(val_v, val_out.at[row])
        pltpu.sync_copy(idx16_v, idx_out.at[row])
        plsc.subcore_barrier()


def _epi_body(val_ref, idx_ref, cnt_ref, dsum_ref, ext_ref, poses_ref,
              meta_ref, out_ref):
    vmax = [jnp.float32(0.0)] * C
    amax = [jnp.int32(0)] * C
    for q in range(4):
        p, co = q // 2, q % 2
        for sl in range(NCLS):
            cl = 1 + p * (2 * NCLS) + co * NCLS + sl
            if cl < C:
                blk = val_ref[pl.ds(q * 16, 16), pl.ds(sl * 16, 16)]
                ibk = idx_ref[pl.ds(q * 16, 16), pl.ds(sl * 16, 16)]
                mv = jnp.max(blk)
                ai = jnp.min(jnp.where(blk == mv, ibk, BIG))
                vmax[cl] = mv
                amax[cl] = ai
    fx = meta_ref[0, 0] * 500.0 + 500.0
    scores = []
    for cl in range(C):
        cnt = cnt_ref[0, cl]
        valid = ((vmax[cl] > 1.0) & (cnt > 500.0)
                 & (vmax[cl] / (cnt + 1.0) > 0.001))
        scores.append(vmax[cl] * valid.astype(jnp.float32))
    tot = scores[0]
    for cl in range(1, C):
        tot = tot + scores[cl]
    for cl in range(C):
        cnt = cnt_ref[0, cl]
        depth = dsum_ref[0, cl] / (cnt + 1e-6)
        e0 = ext_ref[cl, 0]
        e1 = ext_ref[cl, 1]
        e2 = ext_ref[cl, 2]
        diam = jnp.sqrt(e0 * e0 + e1 * e1 + e2 * e2 + 1e-8)
        scale = fx * diam / (jnp.abs(depth) + 0.1)
        cx0 = (amax[cl] % W).astype(jnp.float32)
        cy0 = (amax[cl] // W).astype(jnp.float32)
        out_ref[0, cl, 0] = jnp.float32(0.0)
        out_ref[0, cl, 1] = jnp.float32(float(cl))
        out_ref[0, cl, 2] = cx0 - scale * 0.5
        out_ref[0, cl, 3] = cy0 - scale * 0.5
        out_ref[0, cl, 4] = cx0 + scale * 0.5
        out_ref[0, cl, 5] = cy0 + scale * 0.5
        out_ref[0, cl, 6] = scores[cl]
        pw = scores[cl] / (tot + 1.0)
        for k in range(13):
            out_ref[0, cl, 7 + k] = poses_ref[cl, k] * pw


_epi_call = pl.pallas_call(
    _epi_body,
    out_shape=jax.ShapeDtypeStruct((1, C, 20), jnp.float32),
    in_specs=[
        pl.BlockSpec(memory_space=pltpu.VMEM),
        pl.BlockSpec(memory_space=pltpu.VMEM),
        pl.BlockSpec(memory_space=pltpu.SMEM),
        pl.BlockSpec(memory_space=pltpu.SMEM),
        pl.BlockSpec(memory_space=pltpu.SMEM),
        pl.BlockSpec(memory_space=pltpu.SMEM),
        pl.BlockSpec(memory_space=pltpu.SMEM),
    ],
    out_specs=pl.BlockSpec(memory_space=pltpu.SMEM),
)


@functools.cache
def _sc_vote_call():
    return pl.kernel(
        _sc_vote_body,
        out_type=[
            jax.ShapeDtypeStruct((4 * 16, NCLS * 16), jnp.float32),
            jax.ShapeDtypeStruct((4 * 16, NCLS * 16), jnp.int32),
        ],
        mesh=plsc.VectorSubcoreMesh(core_axis_name="c", subcore_axis_name="s"),
        scratch_types=[
            pltpu.VMEM((CHUNK,), jnp.int32),
            pltpu.VMEM((CHUNK,), jnp.int32),
            pltpu.VMEM((CHUNK,), jnp.float32),
            pltpu.VMEM((CHUNK,), jnp.float32),
            pltpu.VMEM((CHUNK,), jnp.int32),
            pltpu.VMEM((ZCH,), jnp.float32),
            pltpu.VMEM((BCH,), jnp.float32),
            pltpu.VMEM((BCH,), jnp.float32),
            pltpu.VMEM((NCLS * 16,), jnp.float32),
            pltpu.VMEM((NCLS * 16,), jnp.int32),
            pltpu.VMEM_SHARED((NWORDS,), jnp.float32),
            pltpu.SemaphoreType.DMA,
            pltpu.SemaphoreType.DMA,
            pltpu.SemaphoreType.DMA,
            pltpu.SemaphoreType.DMA,
            pltpu.SemaphoreType.DMA,
            pltpu.SemaphoreType.DMA,
        ],
    )


def kernel(label_2d, vertex_pred, extents, poses, meta_data):
    lab_full = label_2d[0].astype(jnp.int32)
    lab_s = jnp.pad(
        lab_full[::SKIP, ::SKIP].reshape(N),
        (0, NP - N), constant_values=C).reshape(NR, NL)
    uvd = _gather_call(vertex_pred[0].reshape(3 * C, H, W).astype(jnp.float32),
                       lab_full)
    uvd_s = jnp.pad(
        uvd[:, ::SKIP, ::SKIP].reshape(3, N),
        ((0, 0), (0, NP - N))).reshape(3, NR, NL)
    bins, wgt, cnt, dsum = _votes_call(uvd_s, lab_s)
    val, idx = _sc_vote_call()(bins.reshape(V), wgt.reshape(V))
    return _epi_call(val, idx, cnt, dsum, extents.astype(jnp.float32),
                     poses.astype(jnp.float32), meta_data.astype(jnp.float32))


# speedup vs baseline: 11.5506x; 1.0079x over previous
import functools

import jax
import jax.numpy as jnp
from jax import lax
from jax.experimental import pallas as pl
from jax.experimental.pallas import tpu as pltpu
from jax.experimental.pallas import tpu_sc as plsc

C = 22
SKIP = 4
STEPS = 64
STEP_LEN = 8.0
H, W = 480, 640
HW = H * W
HS, WS = H // SKIP, W // SKIP
N = HS * WS
NR, NL = 152, 128
NP = NR * NL
V = NP * STEPS

NCLS = 6
NPASS = 2
ACC_MAIN = NCLS * HW
GARB_MASK = 8191
NWORDS = ACC_MAIN + 8192
ZCH = 1024
ZPT = NWORDS // 16
CHUNK = 2048
VPT = V // 16
NCHUNK = VPT // CHUNK
STRIPE = HW // 16
BCH = 800
NBCH = STRIPE // BCH
BIG = 2 ** 30


def _gather_body(vp_ref, lab_ref, uvd_ref):
    lab = lab_ref[...]
    dx = jnp.zeros((8, W), jnp.float32)
    dy = jnp.zeros((8, W), jnp.float32)
    dz = jnp.zeros((8, W), jnp.float32)
    for c in range(C):
        m = lab == c
        dx = dx + jnp.where(m, vp_ref[3 * c + 0], 0.0)
        dy = dy + jnp.where(m, vp_ref[3 * c + 1], 0.0)
        dz = dz + jnp.where(m, vp_ref[3 * c + 2], 0.0)
    nrm = jnp.sqrt(dx * dx + dy * dy) + 1e-6
    uvd_ref[0] = dx / nrm
    uvd_ref[1] = dy / nrm
    uvd_ref[2] = dz


_gather_call = pl.pallas_call(
    _gather_body,
    grid=(H // 8,),
    in_specs=[
        pl.BlockSpec((3 * C, 8, W), lambda g: (0, g, 0)),
        pl.BlockSpec((8, W), lambda g: (g, 0)),
    ],
    out_specs=pl.BlockSpec((3, 8, W), lambda g: (0, g, 0)),
    out_shape=jax.ShapeDtypeStruct((3, H, W), jnp.float32),
)


def _votes_body(uvd_ref, lab_ref, bins_ref, wgt_ref, cnt_ref, dsum_ref):
    lab = lab_ref[...]
    ux = uvd_ref[0]
    uy = uvd_ref[1]
    dz = uvd_ref[2]
    for c in range(C):
        m = lab == c
        cnt_ref[0, c] = jnp.sum(m.astype(jnp.float32))
        dsum_ref[0, c] = jnp.sum(jnp.where(m, dz, 0.0))
    r = lax.broadcasted_iota(jnp.int32, (NR, NL), 0)
    l = lax.broadcasted_iota(jnp.int32, (NR, NL), 1)
    p = r * NL + l
    xs = p % WS
    ys = p // WS
    px = (xs * SKIP).astype(jnp.float32)
    py = (ys * SKIP).astype(jnp.float32)
    labpos = lab > 0
    lab_hw = lab * HW
    for s in range(STEPS):
        t = (s + 1) * STEP_LEN
        cx = px + t * ux
        cy = py + t * uy
        cxr = jnp.clip(jnp.round(cx), 0.0, W - 1.0)
        cyr = jnp.clip(jnp.round(cy), 0.0, H - 1.0)
        wv = (1.0 - jnp.abs(cx - cxr)) * (1.0 - jnp.abs(cy - cyr))
        valid = (cx >= 0) & (cx <= W - 1) & (cy >= 0) & (cy <= H - 1) & labpos
        wv = jnp.clip(wv, 0.0, 1.0) * valid.astype(jnp.float32)
        bins_ref[s] = lab_hw + cyr.astype(jnp.int32) * W + cxr.astype(jnp.int32)
        wgt_ref[s] = wv


_votes_call = pl.pallas_call(
    _votes_body,
    out_shape=[
        jax.ShapeDtypeStruct((STEPS, NR, NL), jnp.int32),
        jax.ShapeDtypeStruct((STEPS, NR, NL), jnp.float32),
        jax.ShapeDtypeStruct((1, C), jnp.float32),
        jax.ShapeDtypeStruct((1, C), jnp.float32),
    ],
    out_specs=[
        pl.BlockSpec(memory_space=pltpu.VMEM),
        pl.BlockSpec(memory_space=pltpu.VMEM),
        pl.BlockSpec(memory_space=pltpu.SMEM),
        pl.BlockSpec(memory_space=pltpu.SMEM),
    ],
)


def _sc_vote_body(bins_hbm, wgt_hbm, val_out, idx_out,
                  bins_v0, bins_v1, wgt_v0, wgt_v1, idx_v,
                  zbuf, mbuf0, mbuf1, val_v, idx16_v, acc,
                  sem_in0, sem_in1, sem_sc, sem_z,
                  sem_mb0, sem_mb1):
    core = lax.axis_index("c")
    sid = lax.axis_index("s")
    lane = lax.iota(jnp.int32, 16)
    bufs = ((bins_v0, wgt_v0, sem_in0),
            (bins_v1, wgt_v1, sem_in1))
    mbufs = ((mbuf0, sem_mb0), (mbuf1, sem_mb1))

    def zb(i, _):
        zbuf[pl.ds(i * 16, 16)] = jnp.zeros((16,), jnp.float32)
        return 0

    lax.fori_loop(0, ZCH // 16, zb, 0)

    for p in range(NPASS):
        base = 1 + p * (2 * NCLS) + core * NCLS
        base_off = base * HW

        pltpu.async_copy(bins_hbm.at[pl.ds(sid * VPT, CHUNK)], bins_v0,
                         sem_in0)
        pltpu.async_copy(wgt_hbm.at[pl.ds(sid * VPT, CHUNK)], wgt_v0,
                         sem_in0)

        if p == 0:
            def zero_body(k, _):
                pltpu.async_copy(zbuf,
                                 acc.at[pl.ds(sid * ZPT + k * ZCH, ZCH)],
                                 sem_z)
                return 0

            lax.fori_loop(0, ZPT // ZCH, zero_body, 0)

            def zero_drain(k, _):
                pltpu.make_async_copy(zbuf, acc.at[pl.ds(sid * ZPT, ZCH)],
                                      sem_z).wait()
                return 0

            lax.fori_loop(0, ZPT // ZCH, zero_drain, 0)
            plsc.subcore_barrier()

        def sc_body(k, _):
            for b in range(2):
                bv, wv, s_in = bufs[b]
                ov, ow, o_in = bufs[1 - b]
                kk = 2 * k + b
                v0 = sid * VPT + kk * CHUNK
                pltpu.make_async_copy(bins_hbm.at[pl.ds(v0, CHUNK)], bv,
                                      s_in).wait()
                pltpu.make_async_copy(wgt_hbm.at[pl.ds(v0, CHUNK)], wv,
                                      s_in).wait()

                @pl.when(kk >= 1)
                def _():
                    pltpu.make_async_copy(ow, acc.at[idx_v], sem_sc).wait()

                @pl.when(kk + 1 < NCHUNK)
                def _():
                    v1 = sid * VPT + (kk + 1) * CHUNK
                    pltpu.async_copy(bins_hbm.at[pl.ds(v1, CHUNK)], ov, o_in)
                    pltpu.async_copy(wgt_hbm.at[pl.ds(v1, CHUNK)], ow, o_in)

                def vb(i, _):
                    bq = bv[pl.ds(i * 16, 16)]
                    rel = bq - base_off
                    ok = (rel >= 0) & (rel < ACC_MAIN)
                    g = ACC_MAIN + (
                        (kk * CHUNK + i * 16 + sid * 2048 + lane) & GARB_MASK)
                    idx_v[pl.ds(i * 16, 16)] = jnp.where(ok, rel, g)
                    return 0

                lax.fori_loop(0, CHUNK // 16, vb, 0)
                pltpu.async_copy(wv, acc.at[idx_v], sem_sc, add=True)
            return 0

        lax.fori_loop(0, NCHUNK // 2, sc_body, 0)
        pltpu.make_async_copy(wgt_v1, acc.at[idx_v], sem_sc).wait()
        plsc.subcore_barrier()

        vinit = jnp.full((16,), -1.0, jnp.float32)
        iinit = jnp.zeros((16,), jnp.int32)
        for j in range(NCLS):
            base_b = j * HW + sid * STRIPE
            pltpu.async_copy(acc.at[pl.ds(base_b, BCH)], mbuf0, sem_mb0)

            def bch2(kb2, carry, base_b=base_b):
                mxc, mic = carry
                for b in range(2):
                    mb, s_mb = mbufs[b]
                    om, o_mb = mbufs[1 - b]
                    kb = 2 * kb2 + b
                    pltpu.make_async_copy(acc.at[pl.ds(base_b, BCH)], mb,
                                          s_mb).wait()

                    @pl.when(kb + 1 < NBCH)
                    def _(kb=kb, om=om, o_mb=o_mb):
                        pltpu.async_copy(
                            acc.at[pl.ds(base_b + (kb + 1) * BCH, BCH)],
                            om, o_mb)

                    if p == 0:
                        pltpu.async_copy(
                            zbuf.at[pl.ds(0, BCH)],
                            acc.at[pl.ds(base_b + kb * BCH, BCH)], sem_z)

                    def vb2(i, c2, kb=kb, mb=mb):
                        mx2, mi2 = c2
                        v = mb[pl.ds(i * 16, 16)]
                        gi = sid * STRIPE + kb * BCH + i * 16 + lane
                        upd = v > mx2
                        return (jnp.where(upd, v, mx2),
                                jnp.where(upd, gi, mi2))

                    mxc, mic = lax.fori_loop(0, BCH // 16, vb2, (mxc, mic))
                return (mxc, mic)

            mx, mi = lax.fori_loop(0, NBCH // 2, bch2, (vinit, iinit))
            val_v[pl.ds(j * 16, 16)] = mx
            idx16_v[pl.ds(j * 16, 16)] = mi
        if p == 0:
            pltpu.async_copy(zbuf.at[pl.ds(0, 512)],
                             acc.at[pl.ds(ACC_MAIN + sid * 512, 512)], sem_z)

            def zdrain(i, _):
                pltpu.make_async_copy(zbuf.at[pl.ds(0, BCH)],
                                      acc.at[pl.ds(0, BCH)], sem_z).wait()
                return 0

            lax.fori_loop(0, NCLS * NBCH, zdrain, 0)
            pltpu.make_async_copy(zbuf.at[pl.ds(0, 512)],
                                  acc.at[pl.ds(0, 512)], sem_z).wait()
        row = (p * 2 + core) * 16 + sid
        pltpu.sync_copy(val_v, val_out.at[row])
        pltpu.sync_copy(idx16_v, idx_out.at[row])
        plsc.subcore_barrier()


def _epi_body(val_ref, idx_ref, cnt_ref, dsum_ref, ext_ref, poses_ref,
              meta_ref, out_ref):
    vmax = [jnp.float32(0.0)] * C
    amax = [jnp.int32(0)] * C
    for q in range(4):
        p, co = q // 2, q % 2
        for sl in range(NCLS):
            cl = 1 + p * (2 * NCLS) + co * NCLS + sl
            if cl < C:
                blk = val_ref[pl.ds(q * 16, 16), pl.ds(sl * 16, 16)]
                ibk = idx_ref[pl.ds(q * 16, 16), pl.ds(sl * 16, 16)]
                mv = jnp.max(blk)
                ai = jnp.min(jnp.where(blk == mv, ibk, BIG))
                vmax[cl] = mv
                amax[cl] = ai
    fx = meta_ref[0, 0] * 500.0 + 500.0
    scores = []
    for cl in range(C):
        cnt = cnt_ref[0, cl]
        valid = ((vmax[cl] > 1.0) & (cnt > 500.0)
                 & (vmax[cl] / (cnt + 1.0) > 0.001))
        scores.append(vmax[cl] * valid.astype(jnp.float32))
    tot = scores[0]
    for cl in range(1, C):
        tot = tot + scores[cl]
    for cl in range(C):
        cnt = cnt_ref[0, cl]
        depth = dsum_ref[0, cl] / (cnt + 1e-6)
        e0 = ext_ref[cl, 0]
        e1 = ext_ref[cl, 1]
        e2 = ext_ref[cl, 2]
        diam = jnp.sqrt(e0 * e0 + e1 * e1 + e2 * e2 + 1e-8)
        scale = fx * diam / (jnp.abs(depth) + 0.1)
        cx0 = (amax[cl] % W).astype(jnp.float32)
        cy0 = (amax[cl] // W).astype(jnp.float32)
        out_ref[0, cl, 0] = jnp.float32(0.0)
        out_ref[0, cl, 1] = jnp.float32(float(cl))
        out_ref[0, cl, 2] = cx0 - scale * 0.5
        out_ref[0, cl, 3] = cy0 - scale * 0.5
        out_ref[0, cl, 4] = cx0 + scale * 0.5
        out_ref[0, cl, 5] = cy0 + scale * 0.5
        out_ref[0, cl, 6] = scores[cl]
        pw = scores[cl] / (tot + 1.0)
        for k in range(13):
            out_ref[0, cl, 7 + k] = poses_ref[cl, k] * pw


_epi_call = pl.pallas_call(
    _epi_body,
    out_shape=jax.ShapeDtypeStruct((1, C, 20), jnp.float32),
    in_specs=[
        pl.BlockSpec(memory_space=pltpu.VMEM),
        pl.BlockSpec(memory_space=pltpu.VMEM),
        pl.BlockSpec(memory_space=pltpu.SMEM),
        pl.BlockSpec(memory_space=pltpu.SMEM),
        pl.BlockSpec(memory_space=pltpu.SMEM),
        pl.BlockSpec(memory_space=pltpu.SMEM),
        pl.BlockSpec(memory_space=pltpu.SMEM),
    ],
    out_specs=pl.BlockSpec(memory_space=pltpu.SMEM),
)


@functools.cache
def _sc_vote_call():
    return pl.kernel(
        _sc_vote_body,
        out_type=[
            jax.ShapeDtypeStruct((4 * 16, NCLS * 16), jnp.float32),
            jax.ShapeDtypeStruct((4 * 16, NCLS * 16), jnp.int32),
        ],
        mesh=plsc.VectorSubcoreMesh(core_axis_name="c", subcore_axis_name="s"),
        scratch_types=[
            pltpu.VMEM((CHUNK,), jnp.int32),
            pltpu.VMEM((CHUNK,), jnp.int32),
            pltpu.VMEM((CHUNK,), jnp.float32),
            pltpu.VMEM((CHUNK,), jnp.float32),
            pltpu.VMEM((CHUNK,), jnp.int32),
            pltpu.VMEM((ZCH,), jnp.float32),
            pltpu.VMEM((BCH,), jnp.float32),
            pltpu.VMEM((BCH,), jnp.float32),
            pltpu.VMEM((NCLS * 16,), jnp.float32),
            pltpu.VMEM((NCLS * 16,), jnp.int32),
            pltpu.VMEM_SHARED((NWORDS,), jnp.float32),
            pltpu.SemaphoreType.DMA,
            pltpu.SemaphoreType.DMA,
            pltpu.SemaphoreType.DMA,
            pltpu.SemaphoreType.DMA,
            pltpu.SemaphoreType.DMA,
            pltpu.SemaphoreType.DMA,
        ],
    )


def kernel(label_2d, vertex_pred, extents, poses, meta_data):
    lab_full = label_2d[0].astype(jnp.int32)
    lab_s = jnp.pad(
        lab_full[::SKIP, ::SKIP].reshape(N),
        (0, NP - N), constant_values=C).reshape(NR, NL)
    uvd = _gather_call(vertex_pred[0].reshape(3 * C, H, W).astype(jnp.float32),
                       lab_full)
    uvd_s = jnp.pad(
        uvd[:, ::SKIP, ::SKIP].reshape(3, N),
        ((0, 0), (0, NP - N))).reshape(3, NR, NL)
    bins, wgt, cnt, dsum = _votes_call(uvd_s, lab_s)
    val, idx = _sc_vote_call()(bins.reshape(V), wgt.reshape(V))
    return _epi_call(val, idx, cnt, dsum, extents.astype(jnp.float32),
                     poses.astype(jnp.float32), meta_data.astype(jnp.float32))
